# Initial kernel scaffold; baseline (speedup 1.0000x reference)
import jax
import jax.numpy as jnp
from jax.experimental import pallas as pl


def kernel(x, edge_index, batch, W1, b1, W2, b2, W3, b3, W4, b4, Wl, bl):
    raise NotImplementedError("write your pallas kernel here")


# speedup vs baseline: 9.0433x; 9.0433x over previous
import functools

import jax
import jax.numpy as jnp
from jax import lax
from jax.experimental import pallas as pl
from jax.experimental.pallas import tpu as pltpu
from jax.experimental.pallas import tpu_sc as plsc

_N = 10000
_E = 320000
_HID = 256
_HALF = 128
_NG = 64

_NC = 2
_NS = 16

_CH = 128
_NCHUNKS = _E // _CH
_CPT = -(-_NCHUNKS // _NS)

_ROWS_PT = 640
_LAST_ROWS = _N - _ROWS_PT * (_NS - 1)

_EPT = _E // (_NC * _NS)
_DCH = 2000
_HROWS = 80
_HR_PT = 8
_HR_TILES = _HROWS // _HR_PT
_BR = 2000
_NB = _N // _BR


def _sc_mesh():
    return plsc.VectorSubcoreMesh(
        core_axis_name="c", subcore_axis_name="s",
        num_cores=_NC, num_subcores=_NS,
    )


_DW = 128
_NW = _NC * _NS
_CPW = -(-_NCHUNKS // _NW)


def _deg_body(dst_hbm, ones_hbm, zdeg_hbm, out0_hbm, out1_hbm,
              didx, onesbuf, acc):
    c = lax.axis_index("c")
    s = lax.axis_index("s")
    wid = c * _NS + s
    r0 = s * _ROWS_PT

    pltpu.sync_copy(ones_hbm, onesbuf)

    @pl.when(s < _NS - 1)
    def _():
        pltpu.sync_copy(zdeg_hbm, acc.at[pl.ds(r0, _ROWS_PT)])

    @pl.when(s == _NS - 1)
    def _():
        pltpu.sync_copy(zdeg_hbm.at[pl.ds(0, _LAST_ROWS)], acc.at[pl.ds(r0, _LAST_ROWS)])

    plsc.subcore_barrier()

    def _chunk(k, carry):
        chunk = k * _NW + wid

        @pl.when(chunk < _NCHUNKS)
        def _():
            pltpu.sync_copy(dst_hbm.at[pl.ds(chunk * _CH, _CH)], didx.at[0])
            pltpu.sync_copy(onesbuf, acc.at[didx.at[0]], add=True)

        return carry

    lax.fori_loop(0, _CPW, _chunk, 0)
    plsc.subcore_barrier()

    def _writeout(out_hbm):
        @pl.when(s < _NS - 1)
        def _():
            pltpu.sync_copy(acc.at[pl.ds(r0, _ROWS_PT)], out_hbm.at[pl.ds(r0, _ROWS_PT)])

        @pl.when(s == _NS - 1)
        def _():
            pltpu.sync_copy(acc.at[pl.ds(r0, _LAST_ROWS)], out_hbm.at[pl.ds(r0, _LAST_ROWS)])

    @pl.when(c == 0)
    def _():
        _writeout(out0_hbm)

    @pl.when(c == 1)
    def _():
        _writeout(out1_hbm)


@functools.cache
def _deg_kernel_fn():
    return pl.kernel(
        _deg_body,
        out_type=(
            jax.ShapeDtypeStruct((_N, _DW), jnp.float32),
            jax.ShapeDtypeStruct((_N, _DW), jnp.float32),
        ),
        mesh=_sc_mesh(),
        scratch_types=[
            pltpu.VMEM((1, _CH), jnp.int32),
            pltpu.VMEM((_CH, _DW), jnp.float32),
            pltpu.VMEM_SHARED((_N, _DW), jnp.float32),
        ],
    )


def _deg_kernel(dst, ones_rows, zdeg):
    return _deg_kernel_fn()(dst, ones_rows, zdeg)


def _agg_body(g0_hbm, g1_hbm, src_hbm, dst_hbm, zrows_hbm,
              out0_hbm, out1_hbm, sidx, didx, buf, acc, gsem):
    c = lax.axis_index("c")
    s = lax.axis_index("s")
    r0 = s * _ROWS_PT

    def _run(g_hbm, out_hbm):
        @pl.when(s < _NS - 1)
        def _():
            pltpu.sync_copy(zrows_hbm, acc.at[pl.ds(r0, _ROWS_PT)])

        @pl.when(s == _NS - 1)
        def _():
            pltpu.sync_copy(zrows_hbm.at[pl.ds(0, _LAST_ROWS)], acc.at[pl.ds(r0, _LAST_ROWS)])

        plsc.subcore_barrier()

        def _chunk(k, carry):
            chunk = k * _NS + s

            @pl.when(chunk < _NCHUNKS)
            def _():
                e0 = chunk * _CH
                pltpu.sync_copy(src_hbm.at[pl.ds(e0, _CH)], sidx)
                pltpu.sync_copy(dst_hbm.at[pl.ds(e0, _CH)], didx.at[0])
                pltpu.async_copy(g_hbm.at[sidx], buf, gsem).wait()
                pltpu.sync_copy(buf, acc.at[didx.at[0]], add=True)

            return carry

        lax.fori_loop(0, _CPT, _chunk, 0)
        plsc.subcore_barrier()

        @pl.when(s < _NS - 1)
        def _():
            pltpu.sync_copy(acc.at[pl.ds(r0, _ROWS_PT)], out_hbm.at[pl.ds(r0, _ROWS_PT)])

        @pl.when(s == _NS - 1)
        def _():
            pltpu.sync_copy(acc.at[pl.ds(r0, _LAST_ROWS)], out_hbm.at[pl.ds(r0, _LAST_ROWS)])

    @pl.when(c == 0)
    def _():
        _run(g0_hbm, out0_hbm)

    @pl.when(c == 1)
    def _():
        _run(g1_hbm, out1_hbm)


@functools.cache
def _agg_kernel_fn():
    return pl.kernel(
        _agg_body,
        out_type=(
            jax.ShapeDtypeStruct((_N, _HALF), jnp.float32),
            jax.ShapeDtypeStruct((_N, _HALF), jnp.float32),
        ),
        mesh=_sc_mesh(),
        scratch_types=[
            pltpu.VMEM((_CH,), jnp.int32),
            pltpu.VMEM((1, _CH), jnp.int32),
            pltpu.VMEM((_CH, _HALF), jnp.float32),
            pltpu.VMEM_SHARED((_N, _HALF), jnp.float32),
            pltpu.SemaphoreType.DMA,
        ],
    )


def _agg_kernel(g0, g1, src, dst, zrows):
    return _agg_kernel_fn()(g0, g1, src, dst, zrows)


def _tc1_body(x_ref, d0_ref, d1_ref, w_ref, g0_ref, g1_ref, dis_ref):
    dis = lax.rsqrt(d0_ref[:, :1] + d1_ref[:, :1] + 1.0)
    h = jnp.dot(x_ref[...], w_ref[...], preferred_element_type=jnp.float32)
    g = h * dis
    g0_ref[...] = g[:, :_HALF]
    g1_ref[...] = g[:, _HALF:]
    dis_ref[...] = dis


def _tc_layer1(x, deg0, deg1, W1):
    return pl.pallas_call(
        _tc1_body,
        grid=(_NB,),
        in_specs=[
            pl.BlockSpec((_BR, _HALF), lambda i: (i, 0)),
            pl.BlockSpec((_BR, _DW), lambda i: (i, 0)),
            pl.BlockSpec((_BR, _DW), lambda i: (i, 0)),
            pl.BlockSpec((_HALF, _HID), lambda i: (0, 0)),
        ],
        out_specs=(
            pl.BlockSpec((_BR, _HALF), lambda i: (i, 0)),
            pl.BlockSpec((_BR, _HALF), lambda i: (i, 0)),
            pl.BlockSpec((_BR, 1), lambda i: (i, 0)),
        ),
        out_shape=(
            jax.ShapeDtypeStruct((_N, _HALF), jnp.float32),
            jax.ShapeDtypeStruct((_N, _HALF), jnp.float32),
            jax.ShapeDtypeStruct((_N, 1), jnp.float32),
        ),
        compiler_params=pltpu.CompilerParams(
            dimension_semantics=("parallel",),
        ),
    )(x, deg0, deg1, W1)


def _tcmid_body(a0_ref, a1_ref, g0_ref, g1_ref, dis_ref, b_ref, w_ref,
                ng0_ref, ng1_ref):
    dis = dis_ref[...]
    b = b_ref[...]
    a0 = jnp.maximum(dis * (a0_ref[...] + g0_ref[...]) + b[:, :_HALF], 0.0)
    a1 = jnp.maximum(dis * (a1_ref[...] + g1_ref[...]) + b[:, _HALF:], 0.0)
    h = (jnp.dot(a0, w_ref[:_HALF, :], preferred_element_type=jnp.float32)
         + jnp.dot(a1, w_ref[_HALF:, :], preferred_element_type=jnp.float32))
    g = h * dis
    ng0_ref[...] = g[:, :_HALF]
    ng1_ref[...] = g[:, _HALF:]


def _tc_layer_mid(agg0, agg1, g0, g1, dis, b_prev, W):
    return pl.pallas_call(
        _tcmid_body,
        grid=(_NB,),
        in_specs=[
            pl.BlockSpec((_BR, _HALF), lambda i: (i, 0)),
            pl.BlockSpec((_BR, _HALF), lambda i: (i, 0)),
            pl.BlockSpec((_BR, _HALF), lambda i: (i, 0)),
            pl.BlockSpec((_BR, _HALF), lambda i: (i, 0)),
            pl.BlockSpec((_BR, 1), lambda i: (i, 0)),
            pl.BlockSpec((1, _HID), lambda i: (0, 0)),
            pl.BlockSpec((_HID, _HID), lambda i: (0, 0)),
        ],
        out_specs=(
            pl.BlockSpec((_BR, _HALF), lambda i: (i, 0)),
            pl.BlockSpec((_BR, _HALF), lambda i: (i, 0)),
        ),
        out_shape=(
            jax.ShapeDtypeStruct((_N, _HALF), jnp.float32),
            jax.ShapeDtypeStruct((_N, _HALF), jnp.float32),
        ),
        compiler_params=pltpu.CompilerParams(
            dimension_semantics=("parallel",),
        ),
    )(agg0, agg1, g0, g1, dis, b_prev, W)


def _tcfin_body(a0_ref, a1_ref, g0_ref, g1_ref, dis_ref, b_ref, batch_ref,
                wl_ref, bl_ref, out_ref):
    i = pl.program_id(0)
    dis = dis_ref[...]
    b = b_ref[...]
    a0 = jnp.maximum(dis * (a0_ref[...] + g0_ref[...]) + b[:, :_HALF], 0.0)
    a1 = jnp.maximum(dis * (a1_ref[...] + g1_ref[...]) + b[:, _HALF:], 0.0)
    sval = (jnp.dot(a0, wl_ref[:_HALF, :], preferred_element_type=jnp.float32)
            + jnp.dot(a1, wl_ref[_HALF:, :], preferred_element_type=jnp.float32))
    bt = batch_ref[0]
    m = (lax.broadcasted_iota(jnp.int32, (_NG, _BR), 0) == bt).astype(jnp.float32)
    contrib = jnp.dot(m, sval, preferred_element_type=jnp.float32)

    @pl.when(i == 0)
    def _():
        out_ref[...] = jnp.broadcast_to(bl_ref[...], (_NG, 1))

    out_ref[...] += contrib


def _tc_final(agg0, agg1, g0, g1, dis, b_prev, batch2d, Wl, bl2d):
    return pl.pallas_call(
        _tcfin_body,
        grid=(_NB,),
        in_specs=[
            pl.BlockSpec((_BR, _HALF), lambda i: (i, 0)),
            pl.BlockSpec((_BR, _HALF), lambda i: (i, 0)),
            pl.BlockSpec((_BR, _HALF), lambda i: (i, 0)),
            pl.BlockSpec((_BR, _HALF), lambda i: (i, 0)),
            pl.BlockSpec((_BR, 1), lambda i: (i, 0)),
            pl.BlockSpec((1, _HID), lambda i: (0, 0)),
            pl.BlockSpec((1, 1, _BR), lambda i: (i, 0, 0)),
            pl.BlockSpec((_HID, 1), lambda i: (0, 0)),
            pl.BlockSpec((1, 1), lambda i: (0, 0)),
        ],
        out_specs=pl.BlockSpec((_NG, 1), lambda i: (0, 0)),
        out_shape=jax.ShapeDtypeStruct((_NG, 1), jnp.float32),
        compiler_params=pltpu.CompilerParams(
            dimension_semantics=("arbitrary",),
        ),
    )(agg0, agg1, g0, g1, dis, b_prev, batch2d, Wl, bl2d)


def kernel(x, edge_index, batch, W1, b1, W2, b2, W3, b3, W4, b4, Wl, bl):
    src = edge_index[0].astype(jnp.int32)
    dst = edge_index[1].astype(jnp.int32)
    batch3d = batch.astype(jnp.int32).reshape(_NB, 1, _BR)

    zrows = jnp.zeros((_ROWS_PT, _HALF), jnp.float32)
    zdeg = zrows
    ones_rows = jnp.ones((_CH, _DW), jnp.float32)

    deg0, deg1 = _deg_kernel(dst, ones_rows, zdeg)
    g0, g1, dis = _tc_layer1(x, deg0, deg1, W1)

    for b_prev, W in ((b1, W2), (b2, W3), (b3, W4)):
        agg0, agg1 = _agg_kernel(g0, g1, src, dst, zrows)
        g0, g1 = _tc_layer_mid(agg0, agg1, g0, g1, dis,
                               b_prev.reshape(1, _HID), W)

    agg0, agg1 = _agg_kernel(g0, g1, src, dst, zrows)
    out = _tc_final(agg0, agg1, g0, g1, dis, b4.reshape(1, _HID), batch3d,
                    Wl, bl.reshape(1, 1))
    return out
